# baseline (speedup 1.0000x reference)
import jax
import jax.numpy as jnp
from jax.experimental import pallas as pl
from jax.experimental.pallas import tpu as pltpu

_NQ = 10
_K = 512
_D = 32
_BLOCK = 4096
_HALF = _BLOCK // 2


def _scores(r, cbT_q, cn_q, iota128):
    rnorm = jnp.sum(r * r, axis=-1, keepdims=True)
    dot = jnp.dot(r.astype(jnp.bfloat16), cbT_q,
                  preferred_element_type=jnp.float32)
    dist = rnorm + dot + cn_q
    d0 = dist[:, 0:128]
    d1 = dist[:, 128:256]
    d2 = dist[:, 256:384]
    d3 = dist[:, 384:512]
    m128 = jnp.minimum(jnp.minimum(d0, d1), jnp.minimum(d2, d3))
    mn = jnp.min(m128, axis=-1, keepdims=True)
    c128 = jnp.where(d0 == m128, 0,
                     jnp.where(d1 == m128, 1,
                               jnp.where(d2 == m128, 2, 3)))
    j128 = iota128 * 4 + c128
    idx = jnp.min(jnp.where(m128 <= mn, j128, _K), axis=-1,
                  keepdims=True)
    return idx


def _lookup(idx, cb3_q, iota):
    onehot = (iota == idx).astype(jnp.bfloat16)
    q3 = jnp.dot(onehot, cb3_q, preferred_element_type=jnp.float32)
    return (q3[:, :_D] + q3[:, _D:2 * _D]) + q3[:, 2 * _D:]


def _rvq_body(z_ref, cbT_ref, cb3_ref, cn_ref, out_ref, idx_ref):
    p = jax.lax.broadcasted_iota(jnp.int32, (_HALF, _K), 1)
    iota = (p & 127) * 4 + (p >> 7)
    iota128 = jax.lax.broadcasted_iota(jnp.int32, (_HALF, 128), 1)
    zA = z_ref[:_HALF]
    zB = z_ref[_HALF:]
    rA, rB = zA, zB
    accA = jnp.zeros_like(zA)
    accB = jnp.zeros_like(zB)
    colsA, colsB = [], []
    for q in range(_NQ):
        cbT_q = cbT_ref[q]
        cn_q = cn_ref[q]
        cb3_q = cb3_ref[q]
        idxA = _scores(rA, cbT_q, cn_q, iota128)
        idxB = _scores(rB, cbT_q, cn_q, iota128)
        quantA = _lookup(idxA, cb3_q, iota)
        quantB = _lookup(idxB, cb3_q, iota)
        rA = rA - quantA
        rB = rB - quantB
        accA = accA + quantA
        accB = accB + quantB
        colsA.append(idxA)
        colsB.append(idxB)
    out_ref[:_HALF] = accA
    out_ref[_HALF:] = accB
    idx_ref[:_HALF] = jnp.concatenate(colsA, axis=-1)
    idx_ref[_HALF:] = jnp.concatenate(colsB, axis=-1)


def kernel(z, codebooks):
    b, n, d = z.shape
    t = b * n
    zf = z.reshape(t, d)
    pos = jnp.arange(_K)
    jperm = (pos % 128) * 4 + pos // 128
    cbp = codebooks[:, jperm, :]
    cbT = jnp.transpose(-2.0 * cbp, (0, 2, 1)).astype(jnp.bfloat16)
    cnorm = jnp.sum(cbp * cbp, axis=-1)[:, None, :]
    hi_f = jax.lax.reduce_precision(cbp, 8, 7)
    r1 = cbp - hi_f
    mid_f = jax.lax.reduce_precision(r1, 8, 7)
    r2 = r1 - mid_f
    cb3 = jnp.concatenate([hi_f.astype(jnp.bfloat16),
                           mid_f.astype(jnp.bfloat16),
                           r2.astype(jnp.bfloat16)], axis=-1)
    out, idx = pl.pallas_call(
        _rvq_body,
        grid=(t // _BLOCK,),
        in_specs=[
            pl.BlockSpec((_BLOCK, d), lambda i: (i, 0)),
            pl.BlockSpec((_NQ, _D, _K), lambda i: (0, 0, 0)),
            pl.BlockSpec((_NQ, _K, 3 * _D), lambda i: (0, 0, 0)),
            pl.BlockSpec((_NQ, 1, _K), lambda i: (0, 0, 0)),
        ],
        out_specs=[
            pl.BlockSpec((_BLOCK, d), lambda i: (i, 0)),
            pl.BlockSpec((_BLOCK, _NQ), lambda i: (i, 0)),
        ],
        out_shape=[
            jax.ShapeDtypeStruct((t, d), z.dtype),
            jax.ShapeDtypeStruct((t, _NQ), jnp.int32),
        ],
        compiler_params=pltpu.CompilerParams(
            dimension_semantics=("parallel",),
        ),
    )(zf, cbT, cb3, cnorm)
    quantized = out.reshape(b, n, d)
    indices = idx.reshape(b, n, _NQ)
    commit_loss = jnp.zeros((_NQ,), dtype=z.dtype)
    return quantized, indices, commit_loss


# speedup vs baseline: 1.2535x; 1.2535x over previous
import jax
import jax.numpy as jnp
from jax.experimental import pallas as pl
from jax.experimental.pallas import tpu as pltpu

_NQ = 10
_K = 512
_D = 32
_BLOCK = 2048
_NSUB = 4
_SUB = _BLOCK // _NSUB


def _scores(r, cbT_q, cn_q, iota128):
    rnorm = jnp.sum(r * r, axis=-1, keepdims=True)
    dot = jnp.dot(r.astype(jnp.bfloat16), cbT_q,
                  preferred_element_type=jnp.float32)
    dist = rnorm + dot + cn_q
    d0 = dist[:, 0:128]
    d1 = dist[:, 128:256]
    d2 = dist[:, 256:384]
    d3 = dist[:, 384:512]
    m128 = jnp.minimum(jnp.minimum(d0, d1), jnp.minimum(d2, d3))
    mn = jnp.min(m128, axis=-1, keepdims=True)
    c128 = jnp.where(d0 == m128, 0,
                     jnp.where(d1 == m128, 1,
                               jnp.where(d2 == m128, 2, 3)))
    j128 = iota128 * 4 + c128
    idx = jnp.min(jnp.where(m128 <= mn, j128, _K), axis=-1,
                  keepdims=True)
    return idx


def _lookup(idx, cb3_q, iota):
    onehot = (iota == idx).astype(jnp.bfloat16)
    q3 = jnp.dot(onehot, cb3_q, preferred_element_type=jnp.float32)
    return (q3[:, :_D] + q3[:, _D:2 * _D]) + q3[:, 2 * _D:]


def _rvq_body(z_ref, cbT_ref, cb3_ref, cn_ref, out_ref, idx_ref):
    p = jax.lax.broadcasted_iota(jnp.int32, (_SUB, _K), 1)
    iota = (p & 127) * 4 + (p >> 7)
    iota128 = jax.lax.broadcasted_iota(jnp.int32, (_SUB, 128), 1)
    rs = [z_ref[s * _SUB:(s + 1) * _SUB] for s in range(_NSUB)]
    accs = [jnp.zeros_like(r) for r in rs]
    cols = [[] for _ in range(_NSUB)]
    for q in range(_NQ):
        cbT_q = cbT_ref[q]
        cn_q = cn_ref[q]
        cb3_q = cb3_ref[q]
        idxs = [_scores(r, cbT_q, cn_q, iota128) for r in rs]
        quants = [_lookup(ix, cb3_q, iota) for ix in idxs]
        rs = [r - qu for r, qu in zip(rs, quants)]
        accs = [a + qu for a, qu in zip(accs, quants)]
        for s in range(_NSUB):
            cols[s].append(idxs[s])
    for s in range(_NSUB):
        out_ref[s * _SUB:(s + 1) * _SUB] = accs[s]
        idx_ref[s * _SUB:(s + 1) * _SUB] = jnp.concatenate(cols[s], axis=-1)


def kernel(z, codebooks):
    b, n, d = z.shape
    t = b * n
    zf = z.reshape(t, d)
    pos = jnp.arange(_K)
    jperm = (pos % 128) * 4 + pos // 128
    cbp = codebooks[:, jperm, :]
    cbT = jnp.transpose(-2.0 * cbp, (0, 2, 1)).astype(jnp.bfloat16)
    cnorm = jnp.sum(cbp * cbp, axis=-1)[:, None, :]
    hi_f = jax.lax.reduce_precision(cbp, 8, 7)
    r1 = cbp - hi_f
    mid_f = jax.lax.reduce_precision(r1, 8, 7)
    r2 = r1 - mid_f
    cb3 = jnp.concatenate([hi_f.astype(jnp.bfloat16),
                           mid_f.astype(jnp.bfloat16),
                           r2.astype(jnp.bfloat16)], axis=-1)
    out, idx = pl.pallas_call(
        _rvq_body,
        grid=(t // _BLOCK,),
        in_specs=[
            pl.BlockSpec((_BLOCK, d), lambda i: (i, 0)),
            pl.BlockSpec((_NQ, _D, _K), lambda i: (0, 0, 0)),
            pl.BlockSpec((_NQ, _K, 3 * _D), lambda i: (0, 0, 0)),
            pl.BlockSpec((_NQ, 1, _K), lambda i: (0, 0, 0)),
        ],
        out_specs=[
            pl.BlockSpec((_BLOCK, d), lambda i: (i, 0)),
            pl.BlockSpec((_BLOCK, _NQ), lambda i: (i, 0)),
        ],
        out_shape=[
            jax.ShapeDtypeStruct((t, d), z.dtype),
            jax.ShapeDtypeStruct((t, _NQ), jnp.int32),
        ],
        compiler_params=pltpu.CompilerParams(
            dimension_semantics=("parallel",),
        ),
    )(zf, cbT, cb3, cnorm)
    quantized = out.reshape(b, n, d)
    indices = idx.reshape(b, n, _NQ)
    commit_loss = jnp.zeros((_NQ,), dtype=z.dtype)
    return quantized, indices, commit_loss


# speedup vs baseline: 1.2604x; 1.0055x over previous
import jax
import jax.numpy as jnp
from jax.experimental import pallas as pl
from jax.experimental.pallas import tpu as pltpu

_NQ = 10
_K = 512
_D = 32
_BLOCK = 2048
_NSUB = 2
_SUB = _BLOCK // _NSUB


def _scores(r, cbT_q, cn_q, iota128):
    rnorm = jnp.sum(r * r, axis=-1, keepdims=True)
    dot = jnp.dot(r.astype(jnp.bfloat16), cbT_q,
                  preferred_element_type=jnp.float32)
    dist = rnorm + dot + cn_q
    d0 = dist[:, 0:128]
    d1 = dist[:, 128:256]
    d2 = dist[:, 256:384]
    d3 = dist[:, 384:512]
    m128 = jnp.minimum(jnp.minimum(d0, d1), jnp.minimum(d2, d3))
    mn = jnp.min(m128, axis=-1, keepdims=True)
    c128 = jnp.where(d0 == m128, 0,
                     jnp.where(d1 == m128, 1,
                               jnp.where(d2 == m128, 2, 3)))
    j128 = iota128 * 4 + c128
    idx = jnp.min(jnp.where(m128 <= mn, j128, _K), axis=-1,
                  keepdims=True)
    return idx


def _lookup(idx, cb3_q, iota):
    onehot = (iota == idx).astype(jnp.bfloat16)
    q3 = jnp.dot(onehot, cb3_q, preferred_element_type=jnp.float32)
    return (q3[:, :_D] + q3[:, _D:2 * _D]) + q3[:, 2 * _D:]


def _rvq_body(z_ref, cbT_ref, cb3_ref, cn_ref, out_ref, idx_ref):
    p = jax.lax.broadcasted_iota(jnp.int32, (_SUB, _K), 1)
    iota = (p & 127) * 4 + (p >> 7)
    iota128 = jax.lax.broadcasted_iota(jnp.int32, (_SUB, 128), 1)
    rs = [z_ref[s * _SUB:(s + 1) * _SUB] for s in range(_NSUB)]
    accs = [jnp.zeros_like(r) for r in rs]
    cols = [[] for _ in range(_NSUB)]
    for q in range(_NQ):
        cbT_q = cbT_ref[q]
        cn_q = cn_ref[q]
        cb3_q = cb3_ref[q]
        idxs = [_scores(r, cbT_q, cn_q, iota128) for r in rs]
        quants = [_lookup(ix, cb3_q, iota) for ix in idxs]
        rs = [r - qu for r, qu in zip(rs, quants)]
        accs = [a + qu for a, qu in zip(accs, quants)]
        for s in range(_NSUB):
            cols[s].append(idxs[s])
    for s in range(_NSUB):
        out_ref[s * _SUB:(s + 1) * _SUB] = accs[s]
        idx_ref[s * _SUB:(s + 1) * _SUB] = jnp.concatenate(cols[s], axis=-1)


def kernel(z, codebooks):
    b, n, d = z.shape
    t = b * n
    zf = z.reshape(t, d)
    pos = jnp.arange(_K)
    jperm = (pos % 128) * 4 + pos // 128
    cbp = codebooks[:, jperm, :]
    cbT = jnp.transpose(-2.0 * cbp, (0, 2, 1)).astype(jnp.bfloat16)
    cnorm = jnp.sum(cbp * cbp, axis=-1)[:, None, :]
    hi_f = jax.lax.reduce_precision(cbp, 8, 7)
    r1 = cbp - hi_f
    mid_f = jax.lax.reduce_precision(r1, 8, 7)
    r2 = r1 - mid_f
    cb3 = jnp.concatenate([hi_f.astype(jnp.bfloat16),
                           mid_f.astype(jnp.bfloat16),
                           r2.astype(jnp.bfloat16)], axis=-1)
    out, idx = pl.pallas_call(
        _rvq_body,
        grid=(t // _BLOCK,),
        in_specs=[
            pl.BlockSpec((_BLOCK, d), lambda i: (i, 0)),
            pl.BlockSpec((_NQ, _D, _K), lambda i: (0, 0, 0)),
            pl.BlockSpec((_NQ, _K, 3 * _D), lambda i: (0, 0, 0)),
            pl.BlockSpec((_NQ, 1, _K), lambda i: (0, 0, 0)),
        ],
        out_specs=[
            pl.BlockSpec((_BLOCK, d), lambda i: (i, 0)),
            pl.BlockSpec((_BLOCK, _NQ), lambda i: (i, 0)),
        ],
        out_shape=[
            jax.ShapeDtypeStruct((t, d), z.dtype),
            jax.ShapeDtypeStruct((t, _NQ), jnp.int32),
        ],
        compiler_params=pltpu.CompilerParams(
            dimension_semantics=("parallel",),
        ),
    )(zf, cbT, cb3, cnorm)
    quantized = out.reshape(b, n, d)
    indices = idx.reshape(b, n, _NQ)
    commit_loss = jnp.zeros((_NQ,), dtype=z.dtype)
    return quantized, indices, commit_loss
